# initial kernel scaffold (unmeasured)
import jax
import jax.numpy as jnp
from jax import lax
from jax.experimental import pallas as pl
from jax.experimental.pallas import tpu as pltpu

N_DEV = 4
SQ = 2048
SKV = 2048
D_MODEL = 1024
H_LOC = 8
DH = 128
HD = H_LOC * DH
QB = 256
N_QB = SQ // QB
CHUNK = SQ // N_DEV
SCALE = 0.08838834764831843
NEG = -1e9


def _body(x_ref, wq_ref, k_ref, v_ref, wo_ref, out_ref,
          rs_buf, send_sems, recv_sems):
    my_i = lax.axis_index("i")

    def qb_body(qb, carry):
        rows = pl.ds(qb * QB, QB)
        q_blk = jnp.dot(x_ref[rows, :], wq_ref[:, :],
                        preferred_element_type=jnp.float32)
        qi = qb * QB + lax.broadcasted_iota(jnp.int32, (QB, SKV), 0)
        ki = lax.broadcasted_iota(jnp.int32, (QB, SKV), 1)
        mask = (jnp.abs(qi - ki) <= 128) | (ki < 32) | (qi < 32)
        ctx_parts = []
        for h in range(H_LOC):
            cols = slice(h * DH, (h + 1) * DH)
            qh = q_blk[:, cols]
            s = lax.dot_general(qh, k_ref[:, cols],
                                (((1,), (1,)), ((), ())),
                                preferred_element_type=jnp.float32) * SCALE
            s = jnp.where(mask, s, NEG)
            m = jnp.max(s, axis=1, keepdims=True)
            w = jnp.exp(s - m)
            p = w / jnp.sum(w, axis=1, keepdims=True)
            ctx_parts.append(jnp.dot(p, v_ref[:, cols],
                                     preferred_element_type=jnp.float32))
        ctx = jnp.concatenate(ctx_parts, axis=1)
        out_ref[rows, :] = jnp.dot(ctx, wo_ref[:, :],
                                   preferred_element_type=jnp.float32)
        return carry

    lax.fori_loop(0, N_QB, qb_body, 0)

    left = lax.rem(my_i + N_DEV - 1, N_DEV)
    right = lax.rem(my_i + 1, N_DEV)

    barrier = pltpu.get_barrier_semaphore()
    for nbr in (left, right):
        pl.semaphore_signal(barrier, inc=1, device_id=(nbr,),
                            device_id_type=pl.DeviceIdType.MESH)
    pl.semaphore_wait(barrier, 2)

    for s in range(N_DEV - 1):
        send_c = lax.rem(my_i + N_DEV - s, N_DEV)
        recv_c = lax.rem(my_i + 2 * N_DEV - s - 1, N_DEV)
        rdma = pltpu.make_async_remote_copy(
            src_ref=out_ref.at[pl.ds(send_c * CHUNK, CHUNK), :],
            dst_ref=rs_buf.at[s],
            send_sem=send_sems.at[s],
            recv_sem=recv_sems.at[s],
            device_id=(right,),
            device_id_type=pl.DeviceIdType.MESH,
        )
        rdma.start()
        rdma.wait()
        acc_rows = pl.ds(recv_c * CHUNK, CHUNK)
        out_ref[acc_rows, :] = out_ref[acc_rows, :] + rs_buf[s]

    for s in range(N_DEV - 1):
        send_c = lax.rem(my_i + 1 + 2 * N_DEV - s, N_DEV)
        chunk_rows = pl.ds(send_c * CHUNK, CHUNK)
        rdma = pltpu.make_async_remote_copy(
            src_ref=out_ref.at[chunk_rows, :],
            dst_ref=out_ref.at[chunk_rows, :],
            send_sem=send_sems.at[N_DEV - 1 + s],
            recv_sem=recv_sems.at[N_DEV - 1 + s],
            device_id=(right,),
            device_id_type=pl.DeviceIdType.MESH,
        )
        rdma.start()
        rdma.wait()


def kernel(x, Wq, K_ext, V_ext, Wo):
    my_i = lax.axis_index("i")
    x2 = x[0]
    k2 = K_ext[0].reshape(SKV, HD)
    v2 = V_ext[0].reshape(SKV, HD)
    wq = lax.dynamic_slice(Wq, (0, my_i * HD), (Wq.shape[0], HD))
    wo = lax.dynamic_slice(Wo, (my_i * HD, 0), (HD, Wo.shape[1]))

    out2 = pl.pallas_call(
        _body,
        out_shape=jax.ShapeDtypeStruct((SQ, D_MODEL), jnp.float32),
        in_specs=[pl.BlockSpec(memory_space=pltpu.VMEM)] * 5,
        out_specs=pl.BlockSpec(memory_space=pltpu.VMEM),
        scratch_shapes=[
            pltpu.VMEM((N_DEV - 1, CHUNK, D_MODEL), jnp.float32),
            pltpu.SemaphoreType.DMA((2 * (N_DEV - 1),)),
            pltpu.SemaphoreType.DMA((2 * (N_DEV - 1),)),
        ],
        compiler_params=pltpu.CompilerParams(collective_id=0),
    )(x2, wq, k2, v2, wo)
    return out2.reshape(1, SQ, D_MODEL)


# baseline (device time: 329409 ns/iter reference)
import jax
import jax.numpy as jnp
from jax import lax
from jax.experimental import pallas as pl
from jax.experimental.pallas import tpu as pltpu

N_DEV = 4
SQ = 2048
SKV = 2048
D_MODEL = 1024
H_LOC = 8
DH = 128
HD = H_LOC * DH
QB = 256
N_QB = SQ // QB
CHUNK = SQ // N_DEV
SCALE = 0.08838834764831843
NEG = -1e9


def _body(q_ref, k_ref, v_ref, wo_ref, out_ref,
          rs_buf, send_sems, recv_sems):
    my_i = lax.axis_index("i")

    def qb_body(qb, carry):
        rows = pl.ds(qb * QB, QB)
        qi = qb * QB + lax.broadcasted_iota(jnp.int32, (QB, SKV), 0)
        ki = lax.broadcasted_iota(jnp.int32, (QB, SKV), 1)
        mask = (jnp.abs(qi - ki) <= 128) | (ki < 32) | (qi < 32)

        def h_body(h, acc):
            qh = q_ref[h, rows, :]
            s = lax.dot_general(qh, k_ref[h],
                                (((1,), (1,)), ((), ())),
                                preferred_element_type=jnp.float32) * SCALE
            s = jnp.where(mask, s, NEG)
            m = jnp.max(s, axis=1, keepdims=True)
            w = jnp.exp(s - m)
            p = w / jnp.sum(w, axis=1, keepdims=True)
            ctx_h = jnp.dot(p, v_ref[h],
                            preferred_element_type=jnp.float32)
            return acc + jnp.dot(ctx_h, wo_ref[h],
                                 preferred_element_type=jnp.float32)

        acc = lax.fori_loop(
            0, H_LOC, h_body, jnp.zeros((QB, D_MODEL), jnp.float32))
        out_ref[rows, :] = acc
        return carry

    lax.fori_loop(0, N_QB, qb_body, 0)

    left = lax.rem(my_i + N_DEV - 1, N_DEV)
    right = lax.rem(my_i + 1, N_DEV)

    barrier = pltpu.get_barrier_semaphore()
    for nbr in (left, right):
        pl.semaphore_signal(barrier, inc=1, device_id=(nbr,),
                            device_id_type=pl.DeviceIdType.MESH)
    pl.semaphore_wait(barrier, 2)

    for s in range(N_DEV - 1):
        send_c = lax.rem(my_i + N_DEV - s, N_DEV)
        recv_c = lax.rem(my_i + 2 * N_DEV - s - 1, N_DEV)
        rdma = pltpu.make_async_remote_copy(
            src_ref=out_ref.at[pl.ds(send_c * CHUNK, CHUNK), :],
            dst_ref=rs_buf.at[s],
            send_sem=send_sems.at[s],
            recv_sem=recv_sems.at[s],
            device_id=(right,),
            device_id_type=pl.DeviceIdType.MESH,
        )
        rdma.start()
        rdma.wait()
        acc_rows = pl.ds(recv_c * CHUNK, CHUNK)
        out_ref[acc_rows, :] = out_ref[acc_rows, :] + rs_buf[s]

    for s in range(N_DEV - 1):
        send_c = lax.rem(my_i + 1 + 2 * N_DEV - s, N_DEV)
        chunk_rows = pl.ds(send_c * CHUNK, CHUNK)
        rdma = pltpu.make_async_remote_copy(
            src_ref=out_ref.at[chunk_rows, :],
            dst_ref=out_ref.at[chunk_rows, :],
            send_sem=send_sems.at[N_DEV - 1 + s],
            recv_sem=recv_sems.at[N_DEV - 1 + s],
            device_id=(right,),
            device_id_type=pl.DeviceIdType.MESH,
        )
        rdma.start()
        rdma.wait()


def kernel(x, Wq, K_ext, V_ext, Wo):
    my_i = lax.axis_index("i")
    wq = lax.dynamic_slice(Wq, (0, my_i * HD), (Wq.shape[0], HD))
    wo = lax.dynamic_slice(Wo, (my_i * HD, 0), (HD, Wo.shape[1]))

    q3 = (x[0] @ wq).reshape(SQ, H_LOC, DH).transpose(1, 0, 2)
    k3 = K_ext[0].transpose(1, 0, 2)
    v3 = V_ext[0].transpose(1, 0, 2)
    wo3 = wo.reshape(H_LOC, DH, D_MODEL)

    out2 = pl.pallas_call(
        _body,
        out_shape=jax.ShapeDtypeStruct((SQ, D_MODEL), jnp.float32),
        in_specs=[pl.BlockSpec(memory_space=pltpu.VMEM)] * 4,
        out_specs=pl.BlockSpec(memory_space=pltpu.VMEM),
        scratch_shapes=[
            pltpu.VMEM((N_DEV - 1, CHUNK, D_MODEL), jnp.float32),
            pltpu.SemaphoreType.DMA((2 * (N_DEV - 1),)),
            pltpu.SemaphoreType.DMA((2 * (N_DEV - 1),)),
        ],
        compiler_params=pltpu.CompilerParams(collective_id=0),
    )(q3, k3, v3, wo3)
    return out2.reshape(1, SQ, D_MODEL)


# device time: 187978 ns/iter; 1.7524x vs baseline; 1.7524x over previous
import jax
import jax.numpy as jnp
from jax import lax
from jax.experimental import pallas as pl
from jax.experimental.pallas import tpu as pltpu

N_DEV = 4
SQ = 2048
SKV = 2048
D_MODEL = 1024
H_LOC = 8
DH = 128
HD = H_LOC * DH
QB = 256
N_QB = SQ // QB
WIN = 512
GB = 128
CHUNK = SQ // N_DEV
SCALE = 0.08838834764831843
NEG = -1e9


def _body(q_ref, k_ref, v_ref, wo_ref, out_ref,
          obf, rs_buf, send_sems, recv_sems):
    my_i = lax.axis_index("i")

    r = lax.broadcasted_iota(jnp.int32, (QB, WIN), 0)
    c = lax.broadcasted_iota(jnp.int32, (QB, WIN), 1)
    mask_glob = lax.broadcasted_iota(jnp.int32, (QB, GB), 1) < 32
    qi0 = lax.broadcasted_iota(jnp.int32, (QB, SKV), 0)
    ki0 = lax.broadcasted_iota(jnp.int32, (QB, SKV), 1)
    mask0 = (jnp.abs(qi0 - ki0) <= 128) | (ki0 < 32) | (qi0 < 32)

    def h0_body(h, acc):
        qh = q_ref[h, :QB, :]
        s = lax.dot_general(qh, k_ref[h], (((1,), (1,)), ((), ())),
                            preferred_element_type=jnp.float32) * SCALE
        s = jnp.where(mask0, s, NEG)
        m = jnp.max(s, axis=1, keepdims=True)
        w = jnp.exp(s - m)
        ctx = jnp.dot(w.astype(jnp.bfloat16), v_ref[h],
                      preferred_element_type=jnp.float32)
        ctx = ctx / jnp.sum(w, axis=1, keepdims=True)
        return acc + jnp.dot(ctx.astype(jnp.bfloat16), wo_ref[h],
                             preferred_element_type=jnp.float32)

    acc0 = lax.fori_loop(0, H_LOC, h0_body,
                         jnp.zeros((QB, D_MODEL), jnp.float32))
    obf[:QB, :] = acc0.astype(jnp.bfloat16)

    def qb_body(qb, carry):
        rows = pl.ds(qb * QB, QB)
        lo = jnp.minimum(qb * QB - 128, SKV - WIN)
        mask_win = jnp.abs(r - c + (qb * QB - lo)) <= 128

        def h_body(h, acc):
            qh = q_ref[h, rows, :]
            kw = k_ref[h, pl.ds(lo, WIN), :]
            sw = lax.dot_general(qh, kw, (((1,), (1,)), ((), ())),
                                 preferred_element_type=jnp.float32) * SCALE
            sw = jnp.where(mask_win, sw, NEG)
            sg = lax.dot_general(qh, k_ref[h, :GB, :],
                                 (((1,), (1,)), ((), ())),
                                 preferred_element_type=jnp.float32) * SCALE
            sg = jnp.where(mask_glob, sg, NEG)
            m = jnp.maximum(jnp.max(sw, axis=1, keepdims=True),
                            jnp.max(sg, axis=1, keepdims=True))
            ww = jnp.exp(sw - m)
            wg = jnp.exp(sg - m)
            denom = (jnp.sum(ww, axis=1, keepdims=True)
                     + jnp.sum(wg, axis=1, keepdims=True))
            ctx = (jnp.dot(ww.astype(jnp.bfloat16), v_ref[h, pl.ds(lo, WIN), :],
                           preferred_element_type=jnp.float32)
                   + jnp.dot(wg.astype(jnp.bfloat16), v_ref[h, :GB, :],
                             preferred_element_type=jnp.float32)) / denom
            return acc + jnp.dot(ctx.astype(jnp.bfloat16), wo_ref[h],
                                 preferred_element_type=jnp.float32)

        acc = lax.fori_loop(0, H_LOC, h_body,
                            jnp.zeros((QB, D_MODEL), jnp.float32))
        obf[rows, :] = acc.astype(jnp.bfloat16)
        return carry

    lax.fori_loop(1, N_QB, qb_body, 0)

    left = lax.rem(my_i + N_DEV - 1, N_DEV)
    right = lax.rem(my_i + 1, N_DEV)

    barrier = pltpu.get_barrier_semaphore()
    for nbr in (left, right):
        pl.semaphore_signal(barrier, inc=1, device_id=(nbr,),
                            device_id_type=pl.DeviceIdType.MESH)
    pl.semaphore_wait(barrier, 2)

    for s in range(N_DEV - 1):
        send_c = lax.rem(my_i + N_DEV - s, N_DEV)
        recv_c = lax.rem(my_i + 2 * N_DEV - s - 1, N_DEV)
        rdma = pltpu.make_async_remote_copy(
            src_ref=obf.at[pl.ds(send_c * CHUNK, CHUNK), :],
            dst_ref=rs_buf.at[s],
            send_sem=send_sems.at[s],
            recv_sem=recv_sems.at[s],
            device_id=(right,),
            device_id_type=pl.DeviceIdType.MESH,
        )
        rdma.start()
        rdma.wait()
        acc_rows = pl.ds(recv_c * CHUNK, CHUNK)
        obf[acc_rows, :] = obf[acc_rows, :] + rs_buf[s]

    for s in range(N_DEV - 1):
        send_c = lax.rem(my_i + 1 + 2 * N_DEV - s, N_DEV)
        chunk_rows = pl.ds(send_c * CHUNK, CHUNK)
        rdma = pltpu.make_async_remote_copy(
            src_ref=obf.at[chunk_rows, :],
            dst_ref=obf.at[chunk_rows, :],
            send_sem=send_sems.at[N_DEV - 1 + s],
            recv_sem=recv_sems.at[N_DEV - 1 + s],
            device_id=(right,),
            device_id_type=pl.DeviceIdType.MESH,
        )
        rdma.start()
        rdma.wait()

    out_ref[:, :] = obf[:, :].astype(jnp.float32)


def kernel(x, Wq, K_ext, V_ext, Wo):
    my_i = lax.axis_index("i")
    wq = lax.dynamic_slice(Wq, (0, my_i * HD), (Wq.shape[0], HD))
    wo = lax.dynamic_slice(Wo, (my_i * HD, 0), (HD, Wo.shape[1]))

    q = x[0].astype(jnp.bfloat16) @ wq.astype(jnp.bfloat16)
    q3 = q.reshape(SQ, H_LOC, DH).transpose(1, 0, 2)
    k3 = K_ext[0].astype(jnp.bfloat16).transpose(1, 0, 2)
    v3 = V_ext[0].astype(jnp.bfloat16).transpose(1, 0, 2)
    wo3 = wo.astype(jnp.bfloat16).reshape(H_LOC, DH, D_MODEL)

    out2 = pl.pallas_call(
        _body,
        out_shape=jax.ShapeDtypeStruct((SQ, D_MODEL), jnp.float32),
        in_specs=[pl.BlockSpec(memory_space=pltpu.VMEM)] * 4,
        out_specs=pl.BlockSpec(memory_space=pltpu.VMEM),
        scratch_shapes=[
            pltpu.VMEM((SQ, D_MODEL), jnp.bfloat16),
            pltpu.VMEM((N_DEV - 1, CHUNK, D_MODEL), jnp.bfloat16),
            pltpu.SemaphoreType.DMA((2 * (N_DEV - 1),)),
            pltpu.SemaphoreType.DMA((2 * (N_DEV - 1),)),
        ],
        compiler_params=pltpu.CompilerParams(collective_id=0),
    )(q3, k3, v3, wo3)
    return out2.reshape(1, SQ, D_MODEL)


# device time: 144504 ns/iter; 2.2796x vs baseline; 1.3008x over previous
import jax
import jax.numpy as jnp
from jax import lax
from jax.experimental import pallas as pl
from jax.experimental.pallas import tpu as pltpu

N_DEV = 4
SQ = 2048
SKV = 2048
D_MODEL = 1024
H_LOC = 8
DH = 128
HD = H_LOC * DH
QB = 256
N_QB = SQ // QB
WIN = 512
GB = 128
GFIX = 32
CHUNK = SQ // N_DEV
SCALE = 0.08838834764831843
NEG = -1e9


def _body(q_ref, k_ref, v_ref, wo_ref, out_ref,
          obf, rs_buf, send_sems, recv_sems):
    my_i = lax.axis_index("i")
    left = lax.rem(my_i + N_DEV - 1, N_DEV)
    right = lax.rem(my_i + 1, N_DEV)

    barrier = pltpu.get_barrier_semaphore()
    for nbr in (left, right):
        pl.semaphore_signal(barrier, inc=1, device_id=(nbr,),
                            device_id_type=pl.DeviceIdType.MESH)
    pl.semaphore_wait(barrier, 2)

    r = lax.broadcasted_iota(jnp.int32, (QB, WIN), 0)
    c = lax.broadcasted_iota(jnp.int32, (QB, WIN), 1)
    cg = lax.broadcasted_iota(jnp.int32, (QB, GB), 1)

    def compute_chunk(chunk_id):
        for b in range(CHUNK // QB):
            qb = chunk_id * (CHUNK // QB) + b
            rows = pl.ds(qb * QB, QB)
            lo = jnp.clip(qb * (QB // 128) - 1, 0, (SKV - WIN) // 128) * 128
            mask_win = (jnp.abs(r - c + (qb * QB - lo)) <= 128) | (lo + c < 32)
            mask_glob = (cg < 32) & (qb > 0)

            def h_body(h, acc):
                qh = q_ref[h, rows, :]
                kw = k_ref[h, pl.ds(lo, WIN), :]
                sw = lax.dot_general(qh, kw, (((1,), (1,)), ((), ())),
                                     preferred_element_type=jnp.float32)
                sw = jnp.exp(jnp.where(mask_win, sw, NEG))
                sg = lax.dot_general(qh, k_ref[h, :GB, :],
                                     (((1,), (1,)), ((), ())),
                                     preferred_element_type=jnp.float32)
                sg = jnp.exp(jnp.where(mask_glob, sg, NEG))
                denom = (jnp.sum(sw, axis=1, keepdims=True)
                         + jnp.sum(sg, axis=1, keepdims=True))
                ctx = (jnp.dot(sw.astype(jnp.bfloat16),
                               v_ref[h, pl.ds(lo, WIN), :],
                               preferred_element_type=jnp.float32)
                       + jnp.dot(sg.astype(jnp.bfloat16), v_ref[h, :GB, :],
                                 preferred_element_type=jnp.float32)) / denom
                return acc + jnp.dot(ctx.astype(jnp.bfloat16), wo_ref[h],
                                     preferred_element_type=jnp.float32)

            acc = lax.fori_loop(0, H_LOC, h_body,
                                jnp.zeros((QB, D_MODEL), jnp.float32))
            obf[rows, :] = acc.astype(jnp.bfloat16)

        @pl.when(chunk_id == 0)
        def _():
            def hfix_body(h, acc):
                s = lax.dot_general(q_ref[h, :GFIX, :], k_ref[h],
                                    (((1,), (1,)), ((), ())),
                                    preferred_element_type=jnp.float32)
                w = jnp.exp(s)
                ctx = jnp.dot(w.astype(jnp.bfloat16), v_ref[h],
                              preferred_element_type=jnp.float32)
                ctx = ctx / jnp.sum(w, axis=1, keepdims=True)
                return acc + jnp.dot(ctx.astype(jnp.bfloat16), wo_ref[h],
                                     preferred_element_type=jnp.float32)

            accf = lax.fori_loop(0, H_LOC, hfix_body,
                                 jnp.zeros((GFIX, D_MODEL), jnp.float32))
            obf[:GFIX, :] = accf.astype(jnp.bfloat16)

    compute_chunk(my_i)
    for s in range(N_DEV - 1):
        send_c = lax.rem(my_i + N_DEV - s, N_DEV)
        recv_c = lax.rem(my_i + 2 * N_DEV - s - 1, N_DEV)
        rdma = pltpu.make_async_remote_copy(
            src_ref=obf.at[pl.ds(send_c * CHUNK, CHUNK), :],
            dst_ref=rs_buf.at[s],
            send_sem=send_sems.at[s],
            recv_sem=recv_sems.at[s],
            device_id=(right,),
            device_id_type=pl.DeviceIdType.MESH,
        )
        rdma.start()
        compute_chunk(recv_c)
        rdma.wait()
        acc_rows = pl.ds(recv_c * CHUNK, CHUNK)
        obf[acc_rows, :] = obf[acc_rows, :] + rs_buf[s]

    for s in range(N_DEV - 1):
        send_c = lax.rem(my_i + 1 + 2 * N_DEV - s, N_DEV)
        chunk_rows = pl.ds(send_c * CHUNK, CHUNK)
        rdma = pltpu.make_async_remote_copy(
            src_ref=obf.at[chunk_rows, :],
            dst_ref=obf.at[chunk_rows, :],
            send_sem=send_sems.at[N_DEV - 1 + s],
            recv_sem=recv_sems.at[N_DEV - 1 + s],
            device_id=(right,),
            device_id_type=pl.DeviceIdType.MESH,
        )
        rdma.start()
        rdma.wait()

    out_ref[:, :] = obf[:, :].astype(jnp.float32)


def kernel(x, Wq, K_ext, V_ext, Wo):
    my_i = lax.axis_index("i")
    wq = lax.dynamic_slice(Wq, (0, my_i * HD), (Wq.shape[0], HD))
    wo = lax.dynamic_slice(Wo, (my_i * HD, 0), (HD, Wo.shape[1]))

    q = (x[0].astype(jnp.bfloat16) @ wq.astype(jnp.bfloat16)) * SCALE
    q3 = q.reshape(SQ, H_LOC, DH).transpose(1, 0, 2)
    k3 = K_ext[0].astype(jnp.bfloat16).transpose(1, 0, 2)
    v3 = V_ext[0].astype(jnp.bfloat16).transpose(1, 0, 2)
    wo3 = wo.astype(jnp.bfloat16).reshape(H_LOC, DH, D_MODEL)

    out2 = pl.pallas_call(
        _body,
        out_shape=jax.ShapeDtypeStruct((SQ, D_MODEL), jnp.float32),
        in_specs=[pl.BlockSpec(memory_space=pltpu.VMEM)] * 4,
        out_specs=pl.BlockSpec(memory_space=pltpu.VMEM),
        scratch_shapes=[
            pltpu.VMEM((SQ, D_MODEL), jnp.bfloat16),
            pltpu.VMEM((N_DEV - 1, CHUNK, D_MODEL), jnp.bfloat16),
            pltpu.SemaphoreType.DMA((2 * (N_DEV - 1),)),
            pltpu.SemaphoreType.DMA((2 * (N_DEV - 1),)),
        ],
        compiler_params=pltpu.CompilerParams(collective_id=0),
    )(q3, k3, v3, wo3)
    return out2.reshape(1, SQ, D_MODEL)


# device time: 133690 ns/iter; 2.4640x vs baseline; 1.0809x over previous
import jax
import jax.numpy as jnp
from jax import lax
from jax.experimental import pallas as pl
from jax.experimental.pallas import tpu as pltpu

N_DEV = 4
SQ = 2048
SKV = 2048
D_MODEL = 1024
H_LOC = 8
DH = 128
HD = H_LOC * DH
QB = 256
N_QB = SQ // QB
WIN = 512
GB = 128
GFIX = 32
CHUNK = SQ // N_DEV
COLH = D_MODEL // 2
SCALE = 0.08838834764831843
NEG = -1e9


def _body(q_ref, k_ref, v_ref, wo_ref, out_ref,
          obf, rs_bufR, rs_bufL, send_sems, recv_sems):
    my_i = lax.axis_index("i")
    left = lax.rem(my_i + N_DEV - 1, N_DEV)
    right = lax.rem(my_i + 1, N_DEV)

    barrier = pltpu.get_barrier_semaphore()
    for nbr in (left, right):
        pl.semaphore_signal(barrier, inc=1, device_id=(nbr,),
                            device_id_type=pl.DeviceIdType.MESH)
    pl.semaphore_wait(barrier, 2)

    r = lax.broadcasted_iota(jnp.int32, (QB, WIN), 0)
    c = lax.broadcasted_iota(jnp.int32, (QB, WIN), 1)
    cg = lax.broadcasted_iota(jnp.int32, (QB, GB), 1)

    def compute_chunk(chunk_id):
        for b in range(CHUNK // QB):
            qb = chunk_id * (CHUNK // QB) + b
            rows = pl.ds(qb * QB, QB)
            lo = jnp.clip(qb * (QB // 128) - 1, 0, (SKV - WIN) // 128) * 128
            mask_win = (jnp.abs(r - c + (qb * QB - lo)) <= 128) | (lo + c < 32)
            mask_glob = (cg < 32) & (qb > 0)

            def h_body(h, acc):
                qh = q_ref[h, rows, :]
                kw = k_ref[h, pl.ds(lo, WIN), :]
                sw = lax.dot_general(qh, kw, (((1,), (1,)), ((), ())),
                                     preferred_element_type=jnp.float32)
                sw = jnp.exp(jnp.where(mask_win, sw.astype(jnp.bfloat16),
                                       jnp.bfloat16(NEG)))
                sg = lax.dot_general(qh, k_ref[h, :GB, :],
                                     (((1,), (1,)), ((), ())),
                                     preferred_element_type=jnp.float32)
                sg = jnp.exp(jnp.where(mask_glob, sg.astype(jnp.bfloat16),
                                       jnp.bfloat16(NEG)))
                denom = (jnp.sum(sw, axis=1, keepdims=True,
                                 dtype=jnp.float32)
                         + jnp.sum(sg, axis=1, keepdims=True,
                                   dtype=jnp.float32))
                ctx = (jnp.dot(sw, v_ref[h, pl.ds(lo, WIN), :],
                               preferred_element_type=jnp.float32)
                       + jnp.dot(sg, v_ref[h, :GB, :],
                                 preferred_element_type=jnp.float32)) / denom
                return acc + jnp.dot(ctx.astype(jnp.bfloat16), wo_ref[h],
                                     preferred_element_type=jnp.float32)

            acc = lax.fori_loop(0, H_LOC, h_body,
                                jnp.zeros((QB, D_MODEL), jnp.float32))
            obf[rows, :] = acc.astype(jnp.bfloat16)

        @pl.when(chunk_id == 0)
        def _():
            def hfix_body(h, acc):
                s = lax.dot_general(q_ref[h, :GFIX, :], k_ref[h],
                                    (((1,), (1,)), ((), ())),
                                    preferred_element_type=jnp.float32)
                w = jnp.exp(s)
                ctx = jnp.dot(w.astype(jnp.bfloat16), v_ref[h],
                              preferred_element_type=jnp.float32)
                ctx = ctx / jnp.sum(w, axis=1, keepdims=True)
                return acc + jnp.dot(ctx.astype(jnp.bfloat16), wo_ref[h],
                                     preferred_element_type=jnp.float32)

            accf = lax.fori_loop(0, H_LOC, hfix_body,
                                 jnp.zeros((GFIX, D_MODEL), jnp.float32))
            obf[:GFIX, :] = accf.astype(jnp.bfloat16)

    def rowsd(ch):
        return pl.ds(lax.rem(ch + 2 * N_DEV, N_DEV) * CHUNK, CHUNK)

    def colsd(is_right):
        return pl.ds(0, COLH) if is_right else pl.ds(COLH, COLH)

    def start_rs(s, ch, is_right):
        r = pltpu.make_async_remote_copy(
            src_ref=obf.at[rowsd(ch), colsd(is_right)],
            dst_ref=(rs_bufR if is_right else rs_bufL).at[s],
            send_sem=send_sems.at[s if is_right else 3 + s],
            recv_sem=recv_sems.at[s if is_right else 3 + s],
            device_id=(right if is_right else left,),
            device_id_type=pl.DeviceIdType.MESH,
        )
        r.start()
        return r

    def acc_rs(s, ch, is_right):
        rr, cc = rowsd(ch), colsd(is_right)
        obf[rr, cc] = obf[rr, cc] + (rs_bufR if is_right else rs_bufL)[s]

    compute_chunk(my_i)
    rR = start_rs(0, my_i, True)
    rL = start_rs(0, my_i, False)
    compute_chunk(lax.rem(my_i + 3, N_DEV))
    rR.wait()
    acc_rs(0, my_i + 3, True)
    rR = start_rs(1, my_i + 3, True)
    compute_chunk(lax.rem(my_i + 1, N_DEV))
    rL.wait()
    acc_rs(0, my_i + 1, False)
    rL = start_rs(1, my_i + 1, False)
    compute_chunk(lax.rem(my_i + 2, N_DEV))
    rR.wait()
    acc_rs(1, my_i + 2, True)
    rR = start_rs(2, my_i + 2, True)
    rL.wait()
    acc_rs(1, my_i + 2, False)
    rL = start_rs(2, my_i + 2, False)
    rR.wait()
    acc_rs(2, my_i + 1, True)
    rL.wait()
    acc_rs(2, my_i + 3, False)

    for s in range(N_DEV - 1):
        agR = pltpu.make_async_remote_copy(
            src_ref=obf.at[rowsd(my_i + 1 - s), colsd(True)],
            dst_ref=obf.at[rowsd(my_i + 1 - s), colsd(True)],
            send_sem=send_sems.at[6 + s],
            recv_sem=recv_sems.at[6 + s],
            device_id=(right,),
            device_id_type=pl.DeviceIdType.MESH,
        )
        agL = pltpu.make_async_remote_copy(
            src_ref=obf.at[rowsd(my_i - 1 + s), colsd(False)],
            dst_ref=obf.at[rowsd(my_i - 1 + s), colsd(False)],
            send_sem=send_sems.at[9 + s],
            recv_sem=recv_sems.at[9 + s],
            device_id=(left,),
            device_id_type=pl.DeviceIdType.MESH,
        )
        agR.start()
        agL.start()
        agR.wait()
        agL.wait()

    out_ref[:, :] = obf[:, :].astype(jnp.float32)


def kernel(x, Wq, K_ext, V_ext, Wo):
    my_i = lax.axis_index("i")
    wq = lax.dynamic_slice(Wq, (0, my_i * HD), (Wq.shape[0], HD))
    wo = lax.dynamic_slice(Wo, (my_i * HD, 0), (HD, Wo.shape[1]))

    q = (x[0].astype(jnp.bfloat16) @ wq.astype(jnp.bfloat16)) * SCALE
    q3 = q.reshape(SQ, H_LOC, DH).transpose(1, 0, 2)
    k3 = K_ext[0].astype(jnp.bfloat16).transpose(1, 0, 2)
    v3 = V_ext[0].astype(jnp.bfloat16).transpose(1, 0, 2)
    wo3 = wo.astype(jnp.bfloat16).reshape(H_LOC, DH, D_MODEL)

    out2 = pl.pallas_call(
        _body,
        out_shape=jax.ShapeDtypeStruct((SQ, D_MODEL), jnp.float32),
        in_specs=[pl.BlockSpec(memory_space=pltpu.VMEM)] * 4,
        out_specs=pl.BlockSpec(memory_space=pltpu.VMEM),
        scratch_shapes=[
            pltpu.VMEM((SQ, D_MODEL), jnp.bfloat16),
            pltpu.VMEM((N_DEV - 1, CHUNK, COLH), jnp.bfloat16),
            pltpu.VMEM((N_DEV - 1, CHUNK, COLH), jnp.bfloat16),
            pltpu.SemaphoreType.DMA((4 * (N_DEV - 1),)),
            pltpu.SemaphoreType.DMA((4 * (N_DEV - 1),)),
        ],
        compiler_params=pltpu.CompilerParams(collective_id=0),
    )(q3, k3, v3, wo3)
    return out2.reshape(1, SQ, D_MODEL)


# device time: 113123 ns/iter; 2.9120x vs baseline; 1.1818x over previous
import jax
import jax.numpy as jnp
from jax import lax
from jax.experimental import pallas as pl
from jax.experimental.pallas import tpu as pltpu

N_DEV = 4
SQ = 2048
SKV = 2048
D_MODEL = 1024
H_LOC = 8
DH = 128
HD = H_LOC * DH
QB = 256
N_QB = SQ // QB
WIN = 512
GB = 128
GFIX = 32
CHUNK = SQ // N_DEV
COLH = D_MODEL // 2
SCALE = 0.08838834764831843
NEG = -1e9


def _body(q_ref, k_ref, v_ref, wo_ref, out_ref,
          obf, ctx_ref, rs_bufR, rs_bufL, send_sems, recv_sems):
    my_i = lax.axis_index("i")
    left = lax.rem(my_i + N_DEV - 1, N_DEV)
    right = lax.rem(my_i + 1, N_DEV)

    barrier = pltpu.get_barrier_semaphore()
    for nbr in (left, right):
        pl.semaphore_signal(barrier, inc=1, device_id=(nbr,),
                            device_id_type=pl.DeviceIdType.MESH)
    pl.semaphore_wait(barrier, 2)

    r = lax.broadcasted_iota(jnp.int32, (QB, WIN), 0)
    c = lax.broadcasted_iota(jnp.int32, (QB, WIN), 1)
    cg = lax.broadcasted_iota(jnp.int32, (QB, GB), 1)

    def compute_chunk(chunk_id):
        for b in range(CHUNK // QB):
            qb = chunk_id * (CHUNK // QB) + b
            rows = pl.ds(qb * QB, QB)
            lo = jnp.clip(qb * (QB // 128) - 1, 0, (SKV - WIN) // 128) * 128
            mask_win = (jnp.abs(r - c + (qb * QB - lo)) <= 128) | (lo + c < 32)
            mask_glob = (cg < 32) & (qb > 0)

            def h_body(h, carry):
                cols = pl.ds(h * DH, DH)
                qh = q_ref[rows, cols]
                kw = k_ref[pl.ds(lo, WIN), cols]
                sw = lax.dot_general(qh, kw, (((1,), (1,)), ((), ())),
                                     preferred_element_type=jnp.float32)
                sw = jnp.exp(jnp.where(mask_win, sw.astype(jnp.bfloat16),
                                       jnp.bfloat16(NEG)))
                sg = lax.dot_general(qh, k_ref[:GB, cols],
                                     (((1,), (1,)), ((), ())),
                                     preferred_element_type=jnp.float32)
                sg = jnp.exp(jnp.where(mask_glob, sg.astype(jnp.bfloat16),
                                       jnp.bfloat16(NEG)))
                denom = (jnp.sum(sw, axis=1, keepdims=True,
                                 dtype=jnp.float32)
                         + jnp.sum(sg, axis=1, keepdims=True,
                                   dtype=jnp.float32))
                ctx = (jnp.dot(sw, v_ref[pl.ds(lo, WIN), cols],
                               preferred_element_type=jnp.float32)
                       + jnp.dot(sg, v_ref[:GB, cols],
                                 preferred_element_type=jnp.float32)) / denom
                ctx_ref[:, cols] = ctx.astype(jnp.bfloat16)
                return carry

            lax.fori_loop(0, H_LOC, h_body, 0)
            acc = jnp.dot(ctx_ref[:, :], wo_ref[:, :],
                          preferred_element_type=jnp.float32)
            obf[rows, :] = acc.astype(jnp.bfloat16)

        @pl.when(chunk_id == 0)
        def _():
            def hfix_body(h, acc):
                cols = pl.ds(h * DH, DH)
                s = lax.dot_general(q_ref[:GFIX, cols], k_ref[:, cols],
                                    (((1,), (1,)), ((), ())),
                                    preferred_element_type=jnp.float32)
                w = jnp.exp(s)
                ctx = jnp.dot(w.astype(jnp.bfloat16), v_ref[:, cols],
                              preferred_element_type=jnp.float32)
                ctx = ctx / jnp.sum(w, axis=1, keepdims=True)
                return acc + jnp.dot(ctx.astype(jnp.bfloat16),
                                     wo_ref[pl.ds(h * DH, DH), :],
                                     preferred_element_type=jnp.float32)

            accf = lax.fori_loop(0, H_LOC, hfix_body,
                                 jnp.zeros((GFIX, D_MODEL), jnp.float32))
            obf[:GFIX, :] = accf.astype(jnp.bfloat16)

    def rowsd(ch):
        return pl.ds(lax.rem(ch + 2 * N_DEV, N_DEV) * CHUNK, CHUNK)

    def colsd(is_right):
        return pl.ds(0, COLH) if is_right else pl.ds(COLH, COLH)

    def start_rs(s, ch, is_right):
        r = pltpu.make_async_remote_copy(
            src_ref=obf.at[rowsd(ch), colsd(is_right)],
            dst_ref=(rs_bufR if is_right else rs_bufL).at[s],
            send_sem=send_sems.at[s if is_right else 3 + s],
            recv_sem=recv_sems.at[s if is_right else 3 + s],
            device_id=(right if is_right else left,),
            device_id_type=pl.DeviceIdType.MESH,
        )
        r.start()
        return r

    def acc_rs(s, ch, is_right):
        rr, cc = rowsd(ch), colsd(is_right)
        obf[rr, cc] = obf[rr, cc] + (rs_bufR if is_right else rs_bufL)[s]

    compute_chunk(my_i)
    rR = start_rs(0, my_i, True)
    rL = start_rs(0, my_i, False)
    compute_chunk(lax.rem(my_i + 3, N_DEV))
    rR.wait()
    acc_rs(0, my_i + 3, True)
    rR = start_rs(1, my_i + 3, True)
    compute_chunk(lax.rem(my_i + 1, N_DEV))
    rL.wait()
    acc_rs(0, my_i + 1, False)
    rL = start_rs(1, my_i + 1, False)
    compute_chunk(lax.rem(my_i + 2, N_DEV))
    rR.wait()
    acc_rs(1, my_i + 2, True)
    rR = start_rs(2, my_i + 2, True)
    rL.wait()
    acc_rs(1, my_i + 2, False)
    rL = start_rs(2, my_i + 2, False)
    rR.wait()
    acc_rs(2, my_i + 1, True)
    rL.wait()
    acc_rs(2, my_i + 3, False)

    for s in range(N_DEV - 1):
        agR = pltpu.make_async_remote_copy(
            src_ref=obf.at[rowsd(my_i + 1 - s), colsd(True)],
            dst_ref=obf.at[rowsd(my_i + 1 - s), colsd(True)],
            send_sem=send_sems.at[6 + s],
            recv_sem=recv_sems.at[6 + s],
            device_id=(right,),
            device_id_type=pl.DeviceIdType.MESH,
        )
        agL = pltpu.make_async_remote_copy(
            src_ref=obf.at[rowsd(my_i - 1 + s), colsd(False)],
            dst_ref=obf.at[rowsd(my_i - 1 + s), colsd(False)],
            send_sem=send_sems.at[9 + s],
            recv_sem=recv_sems.at[9 + s],
            device_id=(left,),
            device_id_type=pl.DeviceIdType.MESH,
        )
        agR.start()
        agL.start()
        agR.wait()
        agL.wait()

    out_ref[:, :] = obf[:, :].astype(jnp.float32)


def kernel(x, Wq, K_ext, V_ext, Wo):
    my_i = lax.axis_index("i")
    wq = lax.dynamic_slice(Wq, (0, my_i * HD), (Wq.shape[0], HD))
    wo = lax.dynamic_slice(Wo, (my_i * HD, 0), (HD, Wo.shape[1]))

    q2 = (x[0].astype(jnp.bfloat16) @ wq.astype(jnp.bfloat16)) * SCALE
    k2 = K_ext[0].reshape(SKV, HD).astype(jnp.bfloat16)
    v2 = V_ext[0].reshape(SKV, HD).astype(jnp.bfloat16)
    wo2 = wo.astype(jnp.bfloat16)

    out2 = pl.pallas_call(
        _body,
        out_shape=jax.ShapeDtypeStruct((SQ, D_MODEL), jnp.float32),
        in_specs=[pl.BlockSpec(memory_space=pltpu.VMEM)] * 4,
        out_specs=pl.BlockSpec(memory_space=pltpu.VMEM),
        scratch_shapes=[
            pltpu.VMEM((SQ, D_MODEL), jnp.bfloat16),
            pltpu.VMEM((QB, HD), jnp.bfloat16),
            pltpu.VMEM((N_DEV - 1, CHUNK, COLH), jnp.bfloat16),
            pltpu.VMEM((N_DEV - 1, CHUNK, COLH), jnp.bfloat16),
            pltpu.SemaphoreType.DMA((4 * (N_DEV - 1),)),
            pltpu.SemaphoreType.DMA((4 * (N_DEV - 1),)),
        ],
        compiler_params=pltpu.CompilerParams(collective_id=0),
    )(q2, k2, v2, wo2)
    return out2.reshape(1, SQ, D_MODEL)
